# baseline (speedup 1.0000x reference)
import functools

import jax
import jax.numpy as jnp
from jax import lax
from jax.experimental import pallas as pl
from jax.experimental.pallas import tpu as pltpu
from jax.experimental.pallas import tpu_sc as plsc

_NB = 4
_NT = 3


@functools.lru_cache(maxsize=None)
def _make_gather(V, D, Bt, H):
    info = plsc.get_sparse_core_info()
    NC, NS, L = info.num_cores, info.num_subcores, info.num_lanes
    NW = NC * NS
    assert Bt % (NW * 128) == 0 and D == L and D % 8 == 0
    bpw = Bt // NW
    nbt = bpw // 128
    neb = D // 8
    stg = neb * nbt * 8 * 128

    mesh = plsc.VectorSubcoreMesh(core_axis_name="c", subcore_axis_name="s")

    @functools.partial(
        pl.kernel,
        mesh=mesh,
        compiler_params=pltpu.CompilerParams(use_tc_tiling_on_sc=False,
                                             needs_layout_passes=False),
        out_type=jax.ShapeDtypeStruct((H * neb * (Bt // 128) * 8 * 128,),
                                      jnp.float32),
        scratch_types=[
            pltpu.VMEM((H, bpw), jnp.int32),
            pltpu.VMEM((_NB, bpw, D), jnp.float32),
            pltpu.VMEM((_NT, stg), jnp.float32),
            pltpu.SemaphoreType.DMA((_NB,)),
            pltpu.SemaphoreType.DMA((_NT,)),
        ],
    )
    def gather_kernel(table_hbm, idxt_hbm, out_hbm,
                      idx_v, rows_v, trows_v, gsem, osem):
        wid = lax.axis_index("s") * NC + lax.axis_index("c")
        base = wid * bpw
        bt0 = wid * nbt

        pltpu.sync_copy(idxt_hbm.at[:, pl.ds(base, bpw)], idx_v)

        def gather_desc(h, b):
            return pltpu.make_async_copy(
                table_hbm.at[idx_v.at[h]], rows_v.at[b], gsem.at[b])

        def out_descs(h, tb):
            return [
                pltpu.make_async_copy(
                    trows_v.at[tb, pl.ds(eb * nbt * 1024, nbt * 1024)],
                    out_hbm.at[pl.ds(((h * neb + eb) * (Bt // 128) + bt0)
                                     * 1024, nbt * 1024)],
                    osem.at[tb])
                for eb in range(neb)
            ]

        ev = lax.iota(jnp.int32, L)
        lane_off = ((ev >> 3) * (nbt * 1024)) + ((ev & 7) * 128)

        def scatter_static(bi, ti):
            def body(r, carry):
                v = rows_v[bi, r, :]
                pos = lane_off + ((r >> 7) * 1024 + (r & 127))
                plsc.store_scatter(trows_v.at[ti], [pos], v)
                return carry
            lax.fori_loop(0, bpw, body, 0, unroll=8)

        def scatter(b, tb):
            for bi in range(_NB):
                for ti in range(_NT):
                    @pl.when(jnp.logical_and(b == bi, tb == ti))
                    def _(bi=bi, ti=ti):
                        scatter_static(bi, ti)

        for t in range(_NB - 1):
            gather_desc(t, t).start()

        def step(h, carry):
            b = lax.rem(h, _NB)
            fb = lax.rem(h + _NB - 1, _NB)
            tb = lax.rem(h, _NT)

            @pl.when(h + _NB - 1 < H)
            def _():
                gather_desc(h + _NB - 1, fb).start()

            gather_desc(h, b).wait()

            @pl.when(h >= _NT)
            def _():
                for c in out_descs(h - _NT, tb):
                    c.wait()

            scatter(b, tb)
            for c in out_descs(h, tb):
                c.start()
            return carry

        lax.fori_loop(0, H, step, 0)
        for c in out_descs(H - 2, lax.rem(H - 2, _NT)):
            c.wait()
        for c in out_descs(H - 1, lax.rem(H - 1, _NT)):
            c.wait()

    return gather_kernel


def kernel(indices, table):
    Bt, H = indices.shape
    V, D = table.shape
    flat = _make_gather(V, D, Bt, H)(table, indices.T)
    z = flat.reshape(H, D // 8, Bt // 128, 8, 128)
    w = jnp.transpose(z, (2, 4, 0, 1, 3))
    return w.reshape(Bt, H, D)


# speedup vs baseline: 1.0064x; 1.0064x over previous
import functools

import jax
import jax.numpy as jnp
from jax import lax
from jax.experimental import pallas as pl
from jax.experimental.pallas import tpu as pltpu
from jax.experimental.pallas import tpu_sc as plsc

_NB = 3
_NT = 2


@functools.lru_cache(maxsize=None)
def _make_gather(V, D, Bt, H):
    info = plsc.get_sparse_core_info()
    NC, NS, L = info.num_cores, info.num_subcores, info.num_lanes
    NW = NC * NS
    assert Bt % (NW * 128) == 0 and D == L and D % 8 == 0
    bpw = Bt // NW
    nbt = bpw // 128
    neb = D // 8
    stg = neb * nbt * 8 * 128

    mesh = plsc.VectorSubcoreMesh(core_axis_name="c", subcore_axis_name="s")

    @functools.partial(
        pl.kernel,
        mesh=mesh,
        compiler_params=pltpu.CompilerParams(use_tc_tiling_on_sc=False,
                                             needs_layout_passes=False),
        out_type=jax.ShapeDtypeStruct((H * neb * (Bt // 128) * 8 * 128,),
                                      jnp.float32),
        scratch_types=[
            pltpu.VMEM((H, bpw), jnp.int32),
            pltpu.VMEM((_NB, bpw, D), jnp.float32),
            pltpu.VMEM((_NT, stg), jnp.float32),
            pltpu.SemaphoreType.DMA((_NB,)),
            pltpu.SemaphoreType.DMA((_NT,)),
        ],
    )
    def gather_kernel(table_hbm, idxt_hbm, out_hbm,
                      idx_v, rows_v, trows_v, gsem, osem):
        wid = lax.axis_index("s") * NC + lax.axis_index("c")
        base = wid * bpw
        bt0 = wid * nbt

        pltpu.sync_copy(idxt_hbm.at[:, pl.ds(base, bpw)], idx_v)

        def gather_desc(h, b):
            return pltpu.make_async_copy(
                table_hbm.at[idx_v.at[h]], rows_v.at[b], gsem.at[b])

        def out_descs(h, tb):
            return [
                pltpu.make_async_copy(
                    trows_v.at[tb, pl.ds(eb * nbt * 1024, nbt * 1024)],
                    out_hbm.at[pl.ds(((h * neb + eb) * (Bt // 128) + bt0)
                                     * 1024, nbt * 1024)],
                    osem.at[tb])
                for eb in range(neb)
            ]

        ev = lax.iota(jnp.int32, L)
        lane_off = ((ev >> 3) * (nbt * 1024)) + ((ev & 7) * 128)

        def scatter_static(bi, ti):
            def body(r, carry):
                v = rows_v[bi, r, :]
                pos = lane_off + ((r >> 7) * 1024 + (r & 127))
                plsc.store_scatter(trows_v.at[ti], [pos], v)
                return carry
            lax.fori_loop(0, bpw, body, 0, unroll=False)

        def scatter(b, tb):
            for bi in range(_NB):
                for ti in range(_NT):
                    @pl.when(jnp.logical_and(b == bi, tb == ti))
                    def _(bi=bi, ti=ti):
                        scatter_static(bi, ti)

        for t in range(_NB - 1):
            gather_desc(t, t).start()

        def step(h, carry):
            b = lax.rem(h, _NB)
            fb = lax.rem(h + _NB - 1, _NB)
            tb = lax.rem(h, _NT)

            @pl.when(h + _NB - 1 < H)
            def _():
                gather_desc(h + _NB - 1, fb).start()

            gather_desc(h, b).wait()

            @pl.when(h >= _NT)
            def _():
                for c in out_descs(h - _NT, tb):
                    c.wait()

            scatter(b, tb)
            for c in out_descs(h, tb):
                c.start()
            return carry

        lax.fori_loop(0, H, step, 0)
        for c in out_descs(H - 2, lax.rem(H - 2, _NT)):
            c.wait()
        for c in out_descs(H - 1, lax.rem(H - 1, _NT)):
            c.wait()

    return gather_kernel


def kernel(indices, table):
    Bt, H = indices.shape
    V, D = table.shape
    flat = _make_gather(V, D, Bt, H)(table, indices.T)
    z = flat.reshape(H, D // 8, Bt // 128, 8, 128)
    w = jnp.transpose(z, (2, 4, 0, 1, 3))
    return w.reshape(Bt, H, D)
